# baseline (device time: 180219 ns/iter reference)
import jax
import jax.numpy as jnp
from jax import lax
from jax.experimental import pallas as pl
from jax.experimental.pallas import tpu as pltpu

N_DEV = 16
M_PER = 256
N_COLS = 2048


def kernel(x, w_mat):
    m_glob, k_per = x.shape
    _, n = w_mat.shape

    half = N_COLS // 2

    def body(x_ref, w_ref, out_ref, xb_ref, wb_ref,
             cw_ref, ccw_ref,
             cw_send, cw_recv, ccw_send, ccw_recv,
             credit_cw, credit_ccw,
             amax_ref, stage_ref, b_send_sems, b_recv_sems):
        my = lax.axis_index("i")
        left = lax.rem(my - 1 + N_DEV, N_DEV)
        right = lax.rem(my + 1, N_DEV)

        barrier = pltpu.get_barrier_semaphore()
        for nbr in (left, right):
            pl.semaphore_signal(barrier, inc=1, device_id=(nbr,),
                                device_id_type=pl.DeviceIdType.MESH)
        pl.semaphore_wait(barrier, 2)

        xb_ref[...] = x_ref[...].astype(jnp.bfloat16)
        wb_ref[...] = w_ref[...].astype(jnp.bfloat16)

        def partial(c, lo):
            return jnp.dot(xb_ref[pl.ds(c * M_PER, M_PER), :],
                           wb_ref[:, lo:lo + half],
                           preferred_element_type=jnp.float32)

        def make_step(t):
            src, dst = t % 2, (t + 1) % 2
            r_cw = pltpu.make_async_remote_copy(
                src_ref=cw_ref.at[src], dst_ref=cw_ref.at[dst],
                send_sem=cw_send.at[src], recv_sem=cw_recv.at[dst],
                device_id=(right,), device_id_type=pl.DeviceIdType.MESH,
            )
            r_ccw = pltpu.make_async_remote_copy(
                src_ref=ccw_ref.at[src], dst_ref=ccw_ref.at[dst],
                send_sem=ccw_send.at[src], recv_sem=ccw_recv.at[dst],
                device_id=(left,), device_id_type=pl.DeviceIdType.MESH,
            )
            r_cw.start()
            r_ccw.start()
            return r_cw, r_ccw

        cw_ref[0] = partial(lax.rem(my - 1 + N_DEV, N_DEV),
                            0).astype(jnp.bfloat16)
        ccw_ref[0] = partial(lax.rem(my + 1, N_DEV),
                             half).astype(jnp.bfloat16)
        r_cw, r_ccw = make_step(0)

        y_l = y_r = None
        for s in range(N_DEV - 1):
            nxt = (s + 1) % 2
            if s < N_DEV - 2:
                p_cw = partial(lax.rem(my - 2 - s + 2 * N_DEV, N_DEV), 0)
                p_ccw = partial(lax.rem(my + 2 + s, N_DEV), half)
            else:
                p_cw = partial(my, 0)
                p_ccw = partial(my, half)
            r_cw.wait()
            r_ccw.wait()
            if s <= N_DEV - 3:
                pl.semaphore_signal(credit_cw, inc=1, device_id=(left,),
                                    device_id_type=pl.DeviceIdType.MESH)
                pl.semaphore_signal(credit_ccw, inc=1, device_id=(right,),
                                    device_id_type=pl.DeviceIdType.MESH)
            if s < N_DEV - 2:
                cw_ref[nxt] = (cw_ref[nxt].astype(jnp.float32)
                               + p_cw).astype(jnp.bfloat16)
                ccw_ref[nxt] = (ccw_ref[nxt].astype(jnp.float32)
                                + p_ccw).astype(jnp.bfloat16)
                pl.semaphore_wait(credit_cw, 1)
                pl.semaphore_wait(credit_ccw, 1)
                r_cw, r_ccw = make_step(s + 1)
            else:
                y_l = cw_ref[nxt].astype(jnp.float32) + p_cw
                y_r = ccw_ref[nxt].astype(jnp.float32) + p_ccw

        y = jnp.maximum(jnp.concatenate([y_l, y_r], axis=1), 0.0)

        amax_ref[...] = jnp.full((8, 128), jnp.max(y), dtype=jnp.float32)
        for k in range(4):
            partner = lax.bitwise_xor(my, 1 << k)
            ex = pltpu.make_async_remote_copy(
                src_ref=amax_ref,
                dst_ref=stage_ref.at[k],
                send_sem=b_send_sems.at[k],
                recv_sem=b_recv_sems.at[k],
                device_id=(partner,),
                device_id_type=pl.DeviceIdType.MESH,
            )
            ex.start()
            ex.wait()
            amax_ref[...] = jnp.maximum(amax_ref[...], stage_ref[k])

        scale = amax_ref[0, 0] / 127.0
        q = jnp.clip(jnp.round(y / scale), -127.0, 127.0)
        out_ref[...] = q * scale

    return pl.pallas_call(
        body,
        out_shape=jax.ShapeDtypeStruct((M_PER, n), jnp.float32),
        in_specs=[pl.BlockSpec(memory_space=pltpu.VMEM),
                  pl.BlockSpec(memory_space=pltpu.VMEM)],
        out_specs=pl.BlockSpec(memory_space=pltpu.VMEM),
        scratch_shapes=[
            pltpu.VMEM((m_glob, k_per), jnp.bfloat16),
            pltpu.VMEM((k_per, N_COLS), jnp.bfloat16),
            pltpu.VMEM((2, M_PER, N_COLS // 2), jnp.bfloat16),
            pltpu.VMEM((2, M_PER, N_COLS // 2), jnp.bfloat16),
            pltpu.SemaphoreType.DMA((2,)),
            pltpu.SemaphoreType.DMA((2,)),
            pltpu.SemaphoreType.DMA((2,)),
            pltpu.SemaphoreType.DMA((2,)),
            pltpu.SemaphoreType.REGULAR,
            pltpu.SemaphoreType.REGULAR,
            pltpu.VMEM((8, 128), jnp.float32),
            pltpu.VMEM((4, 8, 128), jnp.float32),
            pltpu.SemaphoreType.DMA((4,)),
            pltpu.SemaphoreType.DMA((4,)),
        ],
        compiler_params=pltpu.CompilerParams(collective_id=0),
    )(x, w_mat)


# device time: 108489 ns/iter; 1.6612x vs baseline; 1.6612x over previous
import jax
import jax.numpy as jnp
from jax import lax
from jax.experimental import pallas as pl
from jax.experimental.pallas import tpu as pltpu

N_DEV = 16
M_PER = 256
N_COLS = 2048
SUB = 4
SLOTS = 4
SUBW = (N_COLS // 2) // SUB


def kernel(x, w_mat):
    m_glob, k_per = x.shape
    _, n = w_mat.shape

    half = N_COLS // 2

    def body(x_ref, w_ref, out_ref, xb_ref, wb_ref,
             cw_ref, ccw_ref,
             cw_send, cw_recv, ccw_send, ccw_recv,
             credit_cw, credit_ccw,
             amax_ref, stage_ref, b_send_sems, b_recv_sems):
        my = lax.axis_index("i")
        left = lax.rem(my - 1 + N_DEV, N_DEV)
        right = lax.rem(my + 1, N_DEV)

        barrier = pltpu.get_barrier_semaphore()
        for nbr in (left, right):
            pl.semaphore_signal(barrier, inc=1, device_id=(nbr,),
                                device_id_type=pl.DeviceIdType.MESH)
        pl.semaphore_wait(barrier, 2)

        xb_ref[...] = x_ref[...].astype(jnp.bfloat16)
        wb_ref[...] = w_ref[...].astype(jnp.bfloat16)

        def partial(c, lo):
            return jnp.dot(xb_ref[pl.ds(c * M_PER, M_PER), :],
                           wb_ref[:, lo:lo + half],
                           preferred_element_type=jnp.float32)

        def sub_rdma(t, j, dir_ref, dir_send, dir_recv, dst_dev):
            src, dst = t % SLOTS, (t + 1) % SLOTS
            return pltpu.make_async_remote_copy(
                src_ref=dir_ref.at[src, j], dst_ref=dir_ref.at[dst, j],
                send_sem=dir_send.at[src, j], recv_sem=dir_recv.at[dst, j],
                device_id=(dst_dev,), device_id_type=pl.DeviceIdType.MESH,
            )

        def start_step(t):
            rs = []
            for j in range(SUB):
                r_cw = sub_rdma(t, j, cw_ref, cw_send, cw_recv, right)
                r_ccw = sub_rdma(t, j, ccw_ref, ccw_send, ccw_recv, left)
                r_cw.start()
                r_ccw.start()
                rs.append((r_cw, r_ccw))
            return rs

        p_cw = partial(lax.rem(my - 1 + N_DEV, N_DEV), 0)
        p_ccw = partial(lax.rem(my + 1, N_DEV), half)
        for j in range(SUB):
            cw_ref[0, j] = p_cw[:, j * SUBW:(j + 1) * SUBW].astype(jnp.bfloat16)
            ccw_ref[0, j] = p_ccw[:, j * SUBW:(j + 1) * SUBW].astype(jnp.bfloat16)
        flight = start_step(0)

        y_l = y_r = None
        for s in range(N_DEV - 1):
            nxt = (s + 1) % SLOTS
            if s < N_DEV - 2:
                p_cw = partial(lax.rem(my - 2 - s + 2 * N_DEV, N_DEV), 0)
                p_ccw = partial(lax.rem(my + 2 + s, N_DEV), half)
            else:
                p_cw = partial(my, 0)
                p_ccw = partial(my, half)
            if 3 <= s + 1 <= N_DEV - 2:
                pl.semaphore_wait(credit_cw, 1)
                pl.semaphore_wait(credit_ccw, 1)
            nxt_rs = []
            for j, (r_cw, r_ccw) in enumerate(flight):
                r_cw.wait()
                r_ccw.wait()
                lo = j * SUBW
                if s < N_DEV - 2:
                    cw_ref[nxt, j] = (
                        cw_ref[nxt, j].astype(jnp.float32)
                        + p_cw[:, lo:lo + SUBW]).astype(jnp.bfloat16)
                    ccw_ref[nxt, j] = (
                        ccw_ref[nxt, j].astype(jnp.float32)
                        + p_ccw[:, lo:lo + SUBW]).astype(jnp.bfloat16)
                    n_cw = sub_rdma(s + 1, j, cw_ref, cw_send, cw_recv, right)
                    n_ccw = sub_rdma(s + 1, j, ccw_ref, ccw_send, ccw_recv, left)
                    n_cw.start()
                    n_ccw.start()
                    nxt_rs.append((n_cw, n_ccw))
            if s <= N_DEV - SLOTS - 1:
                pl.semaphore_signal(credit_cw, inc=1, device_id=(left,),
                                    device_id_type=pl.DeviceIdType.MESH)
                pl.semaphore_signal(credit_ccw, inc=1, device_id=(right,),
                                    device_id_type=pl.DeviceIdType.MESH)
            if s == N_DEV - 2:
                y_l = jnp.concatenate(
                    [cw_ref[nxt, j] for j in range(SUB)],
                    axis=1).astype(jnp.float32) + p_cw
                y_r = jnp.concatenate(
                    [ccw_ref[nxt, j] for j in range(SUB)],
                    axis=1).astype(jnp.float32) + p_ccw
            flight = nxt_rs

        y = jnp.maximum(jnp.concatenate([y_l, y_r], axis=1), 0.0)

        amax_ref[...] = jnp.full((8, 128), jnp.max(y), dtype=jnp.float32)
        for k in range(4):
            partner = lax.bitwise_xor(my, 1 << k)
            ex = pltpu.make_async_remote_copy(
                src_ref=amax_ref,
                dst_ref=stage_ref.at[k],
                send_sem=b_send_sems.at[k],
                recv_sem=b_recv_sems.at[k],
                device_id=(partner,),
                device_id_type=pl.DeviceIdType.MESH,
            )
            ex.start()
            ex.wait()
            amax_ref[...] = jnp.maximum(amax_ref[...], stage_ref[k])

        scale = amax_ref[0, 0] / 127.0
        q = jnp.clip(jnp.round(y / scale), -127.0, 127.0)
        out_ref[...] = q * scale

    return pl.pallas_call(
        body,
        out_shape=jax.ShapeDtypeStruct((M_PER, n), jnp.float32),
        in_specs=[pl.BlockSpec(memory_space=pltpu.VMEM),
                  pl.BlockSpec(memory_space=pltpu.VMEM)],
        out_specs=pl.BlockSpec(memory_space=pltpu.VMEM),
        scratch_shapes=[
            pltpu.VMEM((m_glob, k_per), jnp.bfloat16),
            pltpu.VMEM((k_per, N_COLS), jnp.bfloat16),
            pltpu.VMEM((SLOTS, SUB, M_PER, SUBW), jnp.bfloat16),
            pltpu.VMEM((SLOTS, SUB, M_PER, SUBW), jnp.bfloat16),
            pltpu.SemaphoreType.DMA((SLOTS, SUB)),
            pltpu.SemaphoreType.DMA((SLOTS, SUB)),
            pltpu.SemaphoreType.DMA((SLOTS, SUB)),
            pltpu.SemaphoreType.DMA((SLOTS, SUB)),
            pltpu.SemaphoreType.REGULAR,
            pltpu.SemaphoreType.REGULAR,
            pltpu.VMEM((8, 128), jnp.float32),
            pltpu.VMEM((4, 8, 128), jnp.float32),
            pltpu.SemaphoreType.DMA((4,)),
            pltpu.SemaphoreType.DMA((4,)),
        ],
        compiler_params=pltpu.CompilerParams(collective_id=0),
    )(x, w_mat)


# device time: 104999 ns/iter; 1.7164x vs baseline; 1.0332x over previous
import jax
import jax.numpy as jnp
from jax import lax
from jax.experimental import pallas as pl
from jax.experimental.pallas import tpu as pltpu

N_DEV = 16
M_PER = 256
N_COLS = 2048
SUB = 4
SLOTS = 4
SUBW = (N_COLS // 2) // SUB


def kernel(x, w_mat):
    m_glob, k_per = x.shape
    _, n = w_mat.shape

    half = N_COLS // 2

    def body(x_ref, w_ref, out_ref, wb_ref,
             cw_ref, ccw_ref,
             cw_send, cw_recv, ccw_send, ccw_recv,
             credit_cw, credit_ccw,
             amax_ref, stage_ref, b_send_sems, b_recv_sems):
        my = lax.axis_index("i")
        left = lax.rem(my - 1 + N_DEV, N_DEV)
        right = lax.rem(my + 1, N_DEV)

        barrier = pltpu.get_barrier_semaphore()
        for nbr in (left, right):
            pl.semaphore_signal(barrier, inc=1, device_id=(nbr,),
                                device_id_type=pl.DeviceIdType.MESH)
        pl.semaphore_wait(barrier, 2)

        wb_ref[...] = w_ref[...].astype(jnp.bfloat16)

        def partial(c, lo):
            xs = x_ref[pl.ds(c * M_PER, M_PER), :].astype(jnp.bfloat16)
            return jnp.dot(xs, wb_ref[:, lo:lo + half],
                           preferred_element_type=jnp.float32)

        def sub_rdma(t, j, dir_ref, dir_send, dir_recv, dst_dev):
            src, dst = t % SLOTS, (t + 1) % SLOTS
            return pltpu.make_async_remote_copy(
                src_ref=dir_ref.at[src, j], dst_ref=dir_ref.at[dst, j],
                send_sem=dir_send.at[src, j], recv_sem=dir_recv.at[dst, j],
                device_id=(dst_dev,), device_id_type=pl.DeviceIdType.MESH,
            )

        def start_step(t):
            rs = []
            for j in range(SUB):
                r_cw = sub_rdma(t, j, cw_ref, cw_send, cw_recv, right)
                r_ccw = sub_rdma(t, j, ccw_ref, ccw_send, ccw_recv, left)
                r_cw.start()
                r_ccw.start()
                rs.append((r_cw, r_ccw))
            return rs

        p_cw = partial(lax.rem(my - 1 + N_DEV, N_DEV), 0)
        p_ccw = partial(lax.rem(my + 1, N_DEV), half)
        for j in range(SUB):
            cw_ref[0, j] = p_cw[:, j * SUBW:(j + 1) * SUBW].astype(jnp.bfloat16)
            ccw_ref[0, j] = p_ccw[:, j * SUBW:(j + 1) * SUBW].astype(jnp.bfloat16)
        flight = start_step(0)

        y_l = y_r = None
        for s in range(N_DEV - 1):
            nxt = (s + 1) % SLOTS
            if s < N_DEV - 2:
                p_cw = partial(lax.rem(my - 2 - s + 2 * N_DEV, N_DEV), 0)
                p_ccw = partial(lax.rem(my + 2 + s, N_DEV), half)
            else:
                p_cw = partial(my, 0)
                p_ccw = partial(my, half)
            if 3 <= s + 1 <= N_DEV - 2:
                pl.semaphore_wait(credit_cw, 1)
                pl.semaphore_wait(credit_ccw, 1)
            nxt_rs = []
            for j, (r_cw, r_ccw) in enumerate(flight):
                r_cw.wait()
                r_ccw.wait()
                lo = j * SUBW
                if s < N_DEV - 2:
                    cw_ref[nxt, j] = (
                        cw_ref[nxt, j].astype(jnp.float32)
                        + p_cw[:, lo:lo + SUBW]).astype(jnp.bfloat16)
                    ccw_ref[nxt, j] = (
                        ccw_ref[nxt, j].astype(jnp.float32)
                        + p_ccw[:, lo:lo + SUBW]).astype(jnp.bfloat16)
                    n_cw = sub_rdma(s + 1, j, cw_ref, cw_send, cw_recv, right)
                    n_ccw = sub_rdma(s + 1, j, ccw_ref, ccw_send, ccw_recv, left)
                    n_cw.start()
                    n_ccw.start()
                    nxt_rs.append((n_cw, n_ccw))
            if s <= N_DEV - SLOTS - 1:
                pl.semaphore_signal(credit_cw, inc=1, device_id=(left,),
                                    device_id_type=pl.DeviceIdType.MESH)
                pl.semaphore_signal(credit_ccw, inc=1, device_id=(right,),
                                    device_id_type=pl.DeviceIdType.MESH)
            if s == N_DEV - 2:
                y_l = jnp.concatenate(
                    [cw_ref[nxt, j] for j in range(SUB)],
                    axis=1).astype(jnp.float32) + p_cw
                y_r = jnp.concatenate(
                    [ccw_ref[nxt, j] for j in range(SUB)],
                    axis=1).astype(jnp.float32) + p_ccw
            flight = nxt_rs

        y = jnp.maximum(jnp.concatenate([y_l, y_r], axis=1), 0.0)

        amax_ref[...] = jnp.full((8, 128), jnp.max(y), dtype=jnp.float32)
        for k, xors in enumerate(((1, 2, 3), (4, 8, 12))):
            exs = []
            for i, xr in enumerate(xors):
                partner = lax.bitwise_xor(my, xr)
                ex = pltpu.make_async_remote_copy(
                    src_ref=amax_ref,
                    dst_ref=stage_ref.at[k, i],
                    send_sem=b_send_sems.at[k, i],
                    recv_sem=b_recv_sems.at[k, i],
                    device_id=(partner,),
                    device_id_type=pl.DeviceIdType.MESH,
                )
                ex.start()
                exs.append(ex)
            m = amax_ref[...]
            for i, ex in enumerate(exs):
                ex.wait()
                m = jnp.maximum(m, stage_ref[k, i])
            amax_ref[...] = m

        scale = amax_ref[0, 0] / 127.0
        q = jnp.clip(jnp.round(y / scale), -127.0, 127.0)
        out_ref[...] = q * scale

    return pl.pallas_call(
        body,
        out_shape=jax.ShapeDtypeStruct((M_PER, n), jnp.float32),
        in_specs=[pl.BlockSpec(memory_space=pltpu.VMEM),
                  pl.BlockSpec(memory_space=pltpu.VMEM)],
        out_specs=pl.BlockSpec(memory_space=pltpu.VMEM),
        scratch_shapes=[
            pltpu.VMEM((k_per, N_COLS), jnp.bfloat16),
            pltpu.VMEM((SLOTS, SUB, M_PER, SUBW), jnp.bfloat16),
            pltpu.VMEM((SLOTS, SUB, M_PER, SUBW), jnp.bfloat16),
            pltpu.SemaphoreType.DMA((SLOTS, SUB)),
            pltpu.SemaphoreType.DMA((SLOTS, SUB)),
            pltpu.SemaphoreType.DMA((SLOTS, SUB)),
            pltpu.SemaphoreType.DMA((SLOTS, SUB)),
            pltpu.SemaphoreType.REGULAR,
            pltpu.SemaphoreType.REGULAR,
            pltpu.VMEM((8, 128), jnp.float32),
            pltpu.VMEM((2, 3, 8, 128), jnp.float32),
            pltpu.SemaphoreType.DMA((2, 3)),
            pltpu.SemaphoreType.DMA((2, 3)),
        ],
        compiler_params=pltpu.CompilerParams(collective_id=0),
    )(x, w_mat)
